# baseline (speedup 1.0000x reference)
import functools

import jax
import jax.numpy as jnp
from jax import lax
from jax.experimental import pallas as pl
from jax.experimental.pallas import tpu as pltpu
from jax.experimental.pallas import tpu_sc as plsc

N = 10000
E = 160000
H = 256
HH = 128
L = 4

EROWS = 1280
EP = EROWS * 128

NC = 2
NS = 16
ROWS_PER_TILE = EROWS // NS
ROWS_PER_WORKER = EROWS // (NC * NS)

TSLICE = 632
TROWS = NS * TSLICE
ZROWS = 312
WSLICE = 624
WREM = N - NS * WSLICE

BLK = 1000
GRID = N // BLK

@functools.lru_cache(maxsize=1)
def _sc_mesh():
    return plsc.VectorSubcoreMesh(core_axis_name="c", subcore_axis_name="s",
                                  num_cores=NC, num_subcores=NS)


def _sc_deg(dst2, z128, o128):
    @functools.partial(
        pl.kernel,
        out_type=(
            jax.ShapeDtypeStruct((N, 128), jnp.float32),
            jax.ShapeDtypeStruct((N, 128), jnp.float32),
        ),
        mesh=_sc_mesh(),
        scratch_types=[
            pltpu.VMEM((128,), jnp.int32),
            pltpu.VMEM((128, 128), jnp.float32),
            pltpu.VMEM_SHARED((TROWS, 128), jnp.float32),
        ],
    )
    def k(d2_hbm, z128_hbm, o128_hbm, dega_hbm, degb_hbm, idx_v, ones_v, table_sh):
        c = lax.axis_index("c")
        s = lax.axis_index("s")
        pltpu.sync_copy(o128_hbm, ones_v)
        pltpu.sync_copy(z128_hbm, table_sh.at[pl.ds(s * TSLICE, TSLICE)])
        plsc.subcore_barrier()
        w = s * NC + c

        @pl.loop(0, ROWS_PER_WORKER)
        def _(j):
            pltpu.sync_copy(d2_hbm.at[w * ROWS_PER_WORKER + j], idx_v)
            pltpu.sync_copy(ones_v, table_sh.at[idx_v], add=True)

        plsc.subcore_barrier()

        @pl.when(c == 0)
        def _():
            pltpu.sync_copy(table_sh.at[pl.ds(s * WSLICE, WSLICE)],
                            dega_hbm.at[pl.ds(s * WSLICE, WSLICE)])

            @pl.when(s == 0)
            def _():
                pltpu.sync_copy(table_sh.at[pl.ds(NS * WSLICE, WREM)],
                                dega_hbm.at[pl.ds(NS * WSLICE, WREM)])

        @pl.when(c == 1)
        def _():
            pltpu.sync_copy(table_sh.at[pl.ds(s * WSLICE, WSLICE)],
                            degb_hbm.at[pl.ds(s * WSLICE, WSLICE)])

            @pl.when(s == 0)
            def _():
                pltpu.sync_copy(table_sh.at[pl.ds(NS * WSLICE, WREM)],
                                degb_hbm.at[pl.ds(NS * WSLICE, WREM)])

    return k(dst2, z128, o128)


def _sc_agg(h0, h1, src2, dst2, z128):
    @functools.partial(
        pl.kernel,
        out_type=(
            jax.ShapeDtypeStruct((N, HH), jnp.float32),
            jax.ShapeDtypeStruct((N, HH), jnp.float32),
        ),
        mesh=_sc_mesh(),
        scratch_types=[
            pltpu.VMEM((ROWS_PER_TILE // 2, 128), jnp.int32),
            pltpu.VMEM((ROWS_PER_TILE // 2, 128), jnp.int32),
            pltpu.VMEM((128, HH), jnp.float32),
            pltpu.VMEM((128, HH), jnp.float32),
            pltpu.VMEM_SHARED((TROWS, HH), jnp.float32),
            pltpu.SemaphoreType.DMA,
            pltpu.SemaphoreType.DMA,
            pltpu.SemaphoreType.DMA,
            pltpu.SemaphoreType.DMA,
        ],
    )
    def k(h0_hbm, h1_hbm, s2_hbm, d2_hbm, z_hbm, a0_hbm, a1_hbm,
          src_v, dst_v, rows_a, rows_b, table_sh, gs_a, gs_b, ss_a, ss_b):
        c = lax.axis_index("c")
        s = lax.axis_index("s")
        base = s * ROWS_PER_TILE
        pltpu.sync_copy(z_hbm, table_sh.at[pl.ds(s * TSLICE, TSLICE)])
        plsc.subcore_barrier()

        def gather(j, buf, sem):
            @pl.when(c == 0)
            def _():
                pltpu.async_copy(h0_hbm.at[src_v.at[j]], buf, sem)

            @pl.when(c == 1)
            def _():
                pltpu.async_copy(h1_hbm.at[src_v.at[j]], buf, sem)

        def wait(buf, sem):
            pltpu.make_async_copy(h0_hbm.at[pl.ds(0, 128)], buf, sem).wait()

        def scatter(j, buf, sem):
            pltpu.async_copy(buf, table_sh.at[dst_v.at[j]], sem, add=True)

        HB = ROWS_PER_TILE // 2
        for half in range(2):
            pltpu.sync_copy(s2_hbm.at[pl.ds(base + HB * half, HB)], src_v)
            pltpu.sync_copy(d2_hbm.at[pl.ds(base + HB * half, HB)], dst_v)
            gather(0, rows_a, gs_a)
            gather(1, rows_b, gs_b)

            @pl.loop(0, HB // 2 - 1)
            def _(p):
                j = 2 * p
                wait(rows_a, gs_a)
                scatter(j, rows_a, ss_a)
                wait(rows_b, gs_b)
                scatter(j + 1, rows_b, ss_b)
                wait(rows_a, ss_a)
                gather(j + 2, rows_a, gs_a)
                wait(rows_b, ss_b)
                gather(j + 3, rows_b, gs_b)

            wait(rows_a, gs_a)
            scatter(HB - 2, rows_a, ss_a)
            wait(rows_b, gs_b)
            scatter(HB - 1, rows_b, ss_b)
            wait(rows_a, ss_a)
            wait(rows_b, ss_b)

        plsc.subcore_barrier()

        @pl.when(c == 0)
        def _():
            pltpu.sync_copy(table_sh.at[pl.ds(s * WSLICE, WSLICE)],
                            a0_hbm.at[pl.ds(s * WSLICE, WSLICE)])

            @pl.when(s == 0)
            def _():
                pltpu.sync_copy(table_sh.at[pl.ds(NS * WSLICE, WREM)],
                                a0_hbm.at[pl.ds(NS * WSLICE, WREM)])

        @pl.when(c == 1)
        def _():
            pltpu.sync_copy(table_sh.at[pl.ds(s * WSLICE, WSLICE)],
                            a1_hbm.at[pl.ds(s * WSLICE, WSLICE)])

            @pl.when(s == 0)
            def _():
                pltpu.sync_copy(table_sh.at[pl.ds(NS * WSLICE, WREM)],
                                a1_hbm.at[pl.ds(NS * WSLICE, WREM)])

    return k(h0, h1, src2, dst2, z128)


def _tc_encoder(aa3, at3, aa_emb_p, at_emb_p, W1, b1r, W2, b2r):
    def body(aa_ref, at_ref, aa_emb_ref, at_emb_ref, w1_ref, b1_ref,
             w2_ref, b2_ref, h0_ref, h1_ref):
        aa = aa_ref[0, 0, :]
        at = at_ref[0, 0, :]
        t_aa = jnp.dot(aa_emb_ref[...], w1_ref[0:32, :],
                       preferred_element_type=jnp.float32)
        t_at = jnp.dot(at_emb_ref[...], w1_ref[32:40, :],
                       preferred_element_type=jnp.float32)
        oh_aa = (lax.broadcasted_iota(jnp.int32, (BLK, 24), 1)
                 == aa[:, None]).astype(jnp.float32)
        oh_at = (lax.broadcasted_iota(jnp.int32, (BLK, 8), 1)
                 == at[:, None]).astype(jnp.float32)
        pre = (jnp.dot(oh_aa, t_aa, preferred_element_type=jnp.float32)
               + jnp.dot(oh_at, t_at, preferred_element_type=jnp.float32)
               + b1_ref[...])
        hmid = jnp.maximum(pre, 0.0)
        h = jnp.dot(hmid, w2_ref[...], preferred_element_type=jnp.float32) + b2_ref[...]
        h0_ref[...] = h[:, :HH]
        h1_ref[...] = h[:, HH:]

    full = lambda *shape: pl.BlockSpec(shape, lambda i: (0,) * len(shape))
    return pl.pallas_call(
        body,
        grid=(GRID,),
        in_specs=[
            pl.BlockSpec((1, 1, BLK), lambda i: (i, 0, 0)),
            pl.BlockSpec((1, 1, BLK), lambda i: (i, 0, 0)),
            full(24, 32),
            full(8, 8),
            full(40, H),
            full(1, H),
            full(H, H),
            full(1, H),
        ],
        out_specs=[
            pl.BlockSpec((BLK, HH), lambda i: (i, 0)),
            pl.BlockSpec((BLK, HH), lambda i: (i, 0)),
        ],
        out_shape=[
            jax.ShapeDtypeStruct((N, HH), jnp.float32),
            jax.ShapeDtypeStruct((N, HH), jnp.float32),
        ],
    )(aa3, at3, aa_emb_p, at_emb_p, W1, b1r, W2, b2r)


def _tc_layer(h0, h1, a0, a1, dega, degb, Wl, blr, Wr, gr, br,
              head_W=None, head_br=None):
    last = head_W is not None

    def body(*refs):
        if last:
            (h0_ref, h1_ref, a0_ref, a1_ref, da_ref, db_ref, wl_ref, bl_ref,
             wr_ref, g_ref, b_ref, hw_ref, hb_ref, h0o_ref, h1o_ref, y_ref) = refs
        else:
            (h0_ref, h1_ref, a0_ref, a1_ref, da_ref, db_ref, wl_ref, bl_ref,
             wr_ref, g_ref, b_ref, h0o_ref, h1o_ref) = refs
        h = jnp.concatenate([h0_ref[...], h1_ref[...]], axis=1)
        agg = jnp.concatenate([a0_ref[...], a1_ref[...]], axis=1)
        deg = da_ref[:, 0:1] + db_ref[:, 0:1]
        deg = jnp.maximum(deg, 1.0)
        agg = agg / deg
        out = (jnp.dot(agg, wl_ref[...], preferred_element_type=jnp.float32)
               + bl_ref[...]
               + jnp.dot(h, wr_ref[...], preferred_element_type=jnp.float32))
        nrm = jnp.sqrt(jnp.sum(out * out, axis=1, keepdims=True))
        out = out / jnp.maximum(nrm, 1e-12)
        y = h + out
        mu = jnp.mean(y, axis=1, keepdims=True)
        var = jnp.mean((y - mu) * (y - mu), axis=1, keepdims=True)
        yn = (y - mu) * lax.rsqrt(var + 1e-5) * g_ref[...] + b_ref[...]
        hn = jnp.maximum(yn, 0.0)
        h0o_ref[...] = hn[:, :HH]
        h1o_ref[...] = hn[:, HH:]
        if last:
            y_ref[...] = (jnp.dot(hn, hw_ref[...],
                                  preferred_element_type=jnp.float32)
                          + hb_ref[...])

    full = lambda *shape: pl.BlockSpec(shape, lambda i: (0,) * len(shape))
    blk2 = lambda w: pl.BlockSpec((BLK, w), lambda i: (i, 0))
    in_specs = [blk2(HH), blk2(HH), blk2(HH), blk2(HH), blk2(HH), blk2(HH),
                full(H, H), full(1, H), full(H, H), full(1, H), full(1, H)]
    out_specs = [blk2(HH), blk2(HH)]
    out_shape = [jax.ShapeDtypeStruct((N, HH), jnp.float32),
                 jax.ShapeDtypeStruct((N, HH), jnp.float32)]
    args = [h0, h1, a0, a1, dega, degb, Wl, blr, Wr, gr, br]
    if last:
        in_specs += [full(H, 8), full(1, 8)]
        out_specs += [blk2(8)]
        out_shape += [jax.ShapeDtypeStruct((N, 8), jnp.float32)]
        args += [head_W, head_br]
    return pl.pallas_call(
        body,
        grid=(GRID,),
        in_specs=in_specs,
        out_specs=out_specs,
        out_shape=out_shape,
    )(*args)


def kernel(aa_idx, atom_idx, edge_index, aa_emb, atom_emb, W1, b1, W2, b2,
           lin_l_W, lin_l_b, lin_r_W, ln_g, ln_b, head_W, head_b):
    aa3 = aa_idx.astype(jnp.int32).reshape(GRID, 1, BLK)
    at3 = atom_idx.astype(jnp.int32).reshape(GRID, 1, BLK)
    aa_emb_p = jnp.pad(aa_emb, ((0, 3), (0, 0)))
    at_emb_p = jnp.pad(atom_emb, ((0, 5), (0, 0)))
    b1r = b1.reshape(1, H)
    b2r = b2.reshape(1, H)
    head_br = head_b.reshape(1, 8)

    src = edge_index[0].astype(jnp.int32)
    dst = edge_index[1].astype(jnp.int32)
    src2 = jnp.concatenate([src, jnp.zeros((EP - E,), jnp.int32)]).reshape(EROWS, 128)
    dst2 = jnp.concatenate([dst, jnp.full((EP - E,), N, jnp.int32)]).reshape(EROWS, 128)
    z128 = jnp.zeros((TSLICE, 128), jnp.float32)
    o128 = jnp.ones((128, 128), jnp.float32)

    dega, degb = _sc_deg(dst2, z128, o128)
    h0, h1 = _tc_encoder(aa3, at3, aa_emb_p, at_emb_p, W1, b1r, W2, b2r)

    for l in range(L - 1):
        a0, a1 = _sc_agg(h0, h1, src2, dst2, z128)
        h0, h1 = _tc_layer(h0, h1, a0, a1, dega, degb,
                           lin_l_W[l], lin_l_b[l].reshape(1, H), lin_r_W[l],
                           ln_g[l].reshape(1, H), ln_b[l].reshape(1, H))
    a0, a1 = _sc_agg(h0, h1, src2, dst2, z128)
    _, _, y = _tc_layer(h0, h1, a0, a1, dega, degb,
                        lin_l_W[L - 1], lin_l_b[L - 1].reshape(1, H),
                        lin_r_W[L - 1], ln_g[L - 1].reshape(1, H),
                        ln_b[L - 1].reshape(1, H), head_W, head_br)
    return y


# speedup vs baseline: 1.0134x; 1.0134x over previous
import functools

import jax
import jax.numpy as jnp
from jax import lax
from jax.experimental import pallas as pl
from jax.experimental.pallas import tpu as pltpu
from jax.experimental.pallas import tpu_sc as plsc

N = 10000
E = 160000
H = 256
HH = 128
L = 4

EROWS = 1280
EP = EROWS * 128

NC = 2
NS = 16
ROWS_PER_TILE = EROWS // NS
ROWS_PER_WORKER = EROWS // (NC * NS)

TSLICE = 632
TROWS = NS * TSLICE
ZROWS = 312
WSLICE = 624
WREM = N - NS * WSLICE

BLK = 1000
GRID = N // BLK

@functools.lru_cache(maxsize=1)
def _sc_mesh():
    return plsc.VectorSubcoreMesh(core_axis_name="c", subcore_axis_name="s",
                                  num_cores=NC, num_subcores=NS)


def _sc_deg(dst2, z128, o128):
    @functools.partial(
        pl.kernel,
        out_type=(
            jax.ShapeDtypeStruct((N, 128), jnp.float32),
            jax.ShapeDtypeStruct((N, 128), jnp.float32),
        ),
        mesh=_sc_mesh(),
        scratch_types=[
            pltpu.VMEM((128,), jnp.int32),
            pltpu.VMEM((128, 128), jnp.float32),
            pltpu.VMEM_SHARED((TROWS, 128), jnp.float32),
        ],
    )
    def k(d2_hbm, z128_hbm, o128_hbm, dega_hbm, degb_hbm, idx_v, ones_v, table_sh):
        c = lax.axis_index("c")
        s = lax.axis_index("s")
        pltpu.sync_copy(o128_hbm, ones_v)
        pltpu.sync_copy(z128_hbm, table_sh.at[pl.ds(s * TSLICE, TSLICE)])
        plsc.subcore_barrier()
        w = s * NC + c

        @pl.loop(0, ROWS_PER_WORKER)
        def _(j):
            pltpu.sync_copy(d2_hbm.at[w * ROWS_PER_WORKER + j], idx_v)
            pltpu.sync_copy(ones_v, table_sh.at[idx_v], add=True)

        plsc.subcore_barrier()

        @pl.when(c == 0)
        def _():
            pltpu.sync_copy(table_sh.at[pl.ds(s * WSLICE, WSLICE)],
                            dega_hbm.at[pl.ds(s * WSLICE, WSLICE)])

            @pl.when(s == 0)
            def _():
                pltpu.sync_copy(table_sh.at[pl.ds(NS * WSLICE, WREM)],
                                dega_hbm.at[pl.ds(NS * WSLICE, WREM)])

        @pl.when(c == 1)
        def _():
            pltpu.sync_copy(table_sh.at[pl.ds(s * WSLICE, WSLICE)],
                            degb_hbm.at[pl.ds(s * WSLICE, WSLICE)])

            @pl.when(s == 0)
            def _():
                pltpu.sync_copy(table_sh.at[pl.ds(NS * WSLICE, WREM)],
                                degb_hbm.at[pl.ds(NS * WSLICE, WREM)])

    return k(dst2, z128, o128)


def _sc_agg(h0, h1, src2, dst2, z128):
    @functools.partial(
        pl.kernel,
        out_type=(
            jax.ShapeDtypeStruct((N, HH), jnp.float32),
            jax.ShapeDtypeStruct((N, HH), jnp.float32),
        ),
        mesh=_sc_mesh(),
        scratch_types=[
            pltpu.VMEM((ROWS_PER_TILE // 2, 128), jnp.int32),
            pltpu.VMEM((ROWS_PER_TILE // 2, 128), jnp.int32),
            pltpu.VMEM((128, HH), jnp.float32),
            pltpu.VMEM((128, HH), jnp.float32),
            pltpu.VMEM_SHARED((TROWS, HH), jnp.float32),
            pltpu.SemaphoreType.DMA,
            pltpu.SemaphoreType.DMA,
        ],
    )
    def k(h0_hbm, h1_hbm, s2_hbm, d2_hbm, z_hbm, a0_hbm, a1_hbm,
          src_v, dst_v, rows_a, rows_b, table_sh, sem_a, sem_b):
        c = lax.axis_index("c")
        s = lax.axis_index("s")
        base = s * ROWS_PER_TILE
        pltpu.sync_copy(z_hbm, table_sh.at[pl.ds(s * TSLICE, TSLICE)])
        plsc.subcore_barrier()

        def gather(j, buf, sem):
            @pl.when(c == 0)
            def _():
                pltpu.async_copy(h0_hbm.at[src_v.at[j]], buf, sem)

            @pl.when(c == 1)
            def _():
                pltpu.async_copy(h1_hbm.at[src_v.at[j]], buf, sem)

        def wait(buf, sem):
            pltpu.make_async_copy(h0_hbm.at[pl.ds(0, 128)], buf, sem).wait()

        def scatter(j, buf):
            pltpu.sync_copy(buf, table_sh.at[dst_v.at[j]], add=True)

        HB = ROWS_PER_TILE // 2
        for half in range(2):
            pltpu.sync_copy(s2_hbm.at[pl.ds(base + HB * half, HB)], src_v)
            pltpu.sync_copy(d2_hbm.at[pl.ds(base + HB * half, HB)], dst_v)
            gather(0, rows_a, sem_a)

            @pl.loop(0, HB // 2 - 1)
            def _(p):
                j = 2 * p
                wait(rows_a, sem_a)
                gather(j + 1, rows_b, sem_b)
                scatter(j, rows_a)
                wait(rows_b, sem_b)
                gather(j + 2, rows_a, sem_a)
                scatter(j + 1, rows_b)

            wait(rows_a, sem_a)
            gather(HB - 1, rows_b, sem_b)
            scatter(HB - 2, rows_a)
            wait(rows_b, sem_b)
            scatter(HB - 1, rows_b)

        plsc.subcore_barrier()

        @pl.when(c == 0)
        def _():
            pltpu.sync_copy(table_sh.at[pl.ds(s * WSLICE, WSLICE)],
                            a0_hbm.at[pl.ds(s * WSLICE, WSLICE)])

            @pl.when(s == 0)
            def _():
                pltpu.sync_copy(table_sh.at[pl.ds(NS * WSLICE, WREM)],
                                a0_hbm.at[pl.ds(NS * WSLICE, WREM)])

        @pl.when(c == 1)
        def _():
            pltpu.sync_copy(table_sh.at[pl.ds(s * WSLICE, WSLICE)],
                            a1_hbm.at[pl.ds(s * WSLICE, WSLICE)])

            @pl.when(s == 0)
            def _():
                pltpu.sync_copy(table_sh.at[pl.ds(NS * WSLICE, WREM)],
                                a1_hbm.at[pl.ds(NS * WSLICE, WREM)])

    return k(h0, h1, src2, dst2, z128)


def _tc_encoder(aa3, at3, aa_emb_p, at_emb_p, W1, b1r, W2, b2r):
    def body(aa_ref, at_ref, aa_emb_ref, at_emb_ref, w1_ref, b1_ref,
             w2_ref, b2_ref, h0_ref, h1_ref):
        aa = aa_ref[0, 0, :]
        at = at_ref[0, 0, :]
        t_aa = jnp.dot(aa_emb_ref[...], w1_ref[0:32, :],
                       preferred_element_type=jnp.float32)
        t_at = jnp.dot(at_emb_ref[...], w1_ref[32:40, :],
                       preferred_element_type=jnp.float32)
        oh_aa = (lax.broadcasted_iota(jnp.int32, (BLK, 24), 1)
                 == aa[:, None]).astype(jnp.float32)
        oh_at = (lax.broadcasted_iota(jnp.int32, (BLK, 8), 1)
                 == at[:, None]).astype(jnp.float32)
        pre = (jnp.dot(oh_aa, t_aa, preferred_element_type=jnp.float32)
               + jnp.dot(oh_at, t_at, preferred_element_type=jnp.float32)
               + b1_ref[...])
        hmid = jnp.maximum(pre, 0.0)
        h = jnp.dot(hmid, w2_ref[...], preferred_element_type=jnp.float32) + b2_ref[...]
        h0_ref[...] = h[:, :HH]
        h1_ref[...] = h[:, HH:]

    full = lambda *shape: pl.BlockSpec(shape, lambda i: (0,) * len(shape))
    return pl.pallas_call(
        body,
        grid=(GRID,),
        in_specs=[
            pl.BlockSpec((1, 1, BLK), lambda i: (i, 0, 0)),
            pl.BlockSpec((1, 1, BLK), lambda i: (i, 0, 0)),
            full(24, 32),
            full(8, 8),
            full(40, H),
            full(1, H),
            full(H, H),
            full(1, H),
        ],
        out_specs=[
            pl.BlockSpec((BLK, HH), lambda i: (i, 0)),
            pl.BlockSpec((BLK, HH), lambda i: (i, 0)),
        ],
        out_shape=[
            jax.ShapeDtypeStruct((N, HH), jnp.float32),
            jax.ShapeDtypeStruct((N, HH), jnp.float32),
        ],
    )(aa3, at3, aa_emb_p, at_emb_p, W1, b1r, W2, b2r)


def _tc_layer(h0, h1, a0, a1, dega, degb, Wl, blr, Wr, gr, br,
              head_W=None, head_br=None):
    last = head_W is not None

    def body(*refs):
        if last:
            (h0_ref, h1_ref, a0_ref, a1_ref, da_ref, db_ref, wl_ref, bl_ref,
             wr_ref, g_ref, b_ref, hw_ref, hb_ref, h0o_ref, h1o_ref, y_ref) = refs
        else:
            (h0_ref, h1_ref, a0_ref, a1_ref, da_ref, db_ref, wl_ref, bl_ref,
             wr_ref, g_ref, b_ref, h0o_ref, h1o_ref) = refs
        h = jnp.concatenate([h0_ref[...], h1_ref[...]], axis=1)
        agg = jnp.concatenate([a0_ref[...], a1_ref[...]], axis=1)
        deg = da_ref[:, 0:1] + db_ref[:, 0:1]
        deg = jnp.maximum(deg, 1.0)
        agg = agg / deg
        out = (jnp.dot(agg, wl_ref[...], preferred_element_type=jnp.float32)
               + bl_ref[...]
               + jnp.dot(h, wr_ref[...], preferred_element_type=jnp.float32))
        nrm = jnp.sqrt(jnp.sum(out * out, axis=1, keepdims=True))
        out = out / jnp.maximum(nrm, 1e-12)
        y = h + out
        mu = jnp.mean(y, axis=1, keepdims=True)
        var = jnp.mean((y - mu) * (y - mu), axis=1, keepdims=True)
        yn = (y - mu) * lax.rsqrt(var + 1e-5) * g_ref[...] + b_ref[...]
        hn = jnp.maximum(yn, 0.0)
        h0o_ref[...] = hn[:, :HH]
        h1o_ref[...] = hn[:, HH:]
        if last:
            y_ref[...] = (jnp.dot(hn, hw_ref[...],
                                  preferred_element_type=jnp.float32)
                          + hb_ref[...])

    full = lambda *shape: pl.BlockSpec(shape, lambda i: (0,) * len(shape))
    blk2 = lambda w: pl.BlockSpec((BLK, w), lambda i: (i, 0))
    in_specs = [blk2(HH), blk2(HH), blk2(HH), blk2(HH), blk2(HH), blk2(HH),
                full(H, H), full(1, H), full(H, H), full(1, H), full(1, H)]
    out_specs = [blk2(HH), blk2(HH)]
    out_shape = [jax.ShapeDtypeStruct((N, HH), jnp.float32),
                 jax.ShapeDtypeStruct((N, HH), jnp.float32)]
    args = [h0, h1, a0, a1, dega, degb, Wl, blr, Wr, gr, br]
    if last:
        in_specs += [full(H, 8), full(1, 8)]
        out_specs += [blk2(8)]
        out_shape += [jax.ShapeDtypeStruct((N, 8), jnp.float32)]
        args += [head_W, head_br]
    return pl.pallas_call(
        body,
        grid=(GRID,),
        in_specs=in_specs,
        out_specs=out_specs,
        out_shape=out_shape,
    )(*args)


def kernel(aa_idx, atom_idx, edge_index, aa_emb, atom_emb, W1, b1, W2, b2,
           lin_l_W, lin_l_b, lin_r_W, ln_g, ln_b, head_W, head_b):
    aa3 = aa_idx.astype(jnp.int32).reshape(GRID, 1, BLK)
    at3 = atom_idx.astype(jnp.int32).reshape(GRID, 1, BLK)
    aa_emb_p = jnp.pad(aa_emb, ((0, 3), (0, 0)))
    at_emb_p = jnp.pad(atom_emb, ((0, 5), (0, 0)))
    b1r = b1.reshape(1, H)
    b2r = b2.reshape(1, H)
    head_br = head_b.reshape(1, 8)

    src = edge_index[0].astype(jnp.int32)
    dst = edge_index[1].astype(jnp.int32)
    src2 = jnp.concatenate([src, jnp.zeros((EP - E,), jnp.int32)]).reshape(EROWS, 128)
    dst2 = jnp.concatenate([dst, jnp.full((EP - E,), N, jnp.int32)]).reshape(EROWS, 128)
    z128 = jnp.zeros((TSLICE, 128), jnp.float32)
    o128 = jnp.ones((128, 128), jnp.float32)

    dega, degb = _sc_deg(dst2, z128, o128)
    h0, h1 = _tc_encoder(aa3, at3, aa_emb_p, at_emb_p, W1, b1r, W2, b2r)

    for l in range(L - 1):
        a0, a1 = _sc_agg(h0, h1, src2, dst2, z128)
        h0, h1 = _tc_layer(h0, h1, a0, a1, dega, degb,
                           lin_l_W[l], lin_l_b[l].reshape(1, H), lin_r_W[l],
                           ln_g[l].reshape(1, H), ln_b[l].reshape(1, H))
    a0, a1 = _sc_agg(h0, h1, src2, dst2, z128)
    _, _, y = _tc_layer(h0, h1, a0, a1, dega, degb,
                        lin_l_W[L - 1], lin_l_b[L - 1].reshape(1, H),
                        lin_r_W[L - 1], ln_g[L - 1].reshape(1, H),
                        ln_b[L - 1].reshape(1, H), head_W, head_br)
    return y
